# baseline (speedup 1.0000x reference)
import functools

import jax
import jax.numpy as jnp
from jax.experimental import pallas as pl

N = 32768
DIM = 64
K = 1024
BN = 2048


def _assign_kernel(x_ref, c_ref, out_ref):
    x = x_ref[...]
    c = c_ref[...]
    dot = jax.lax.dot_general(
        x, c, (((1,), (1,)), ((), ())),
        preferred_element_type=jnp.float32)
    x2 = jnp.sum(x * x, axis=1)
    c2 = jnp.sum(c * c, axis=1)
    d2 = x2[:, None] - 2.0 * dot + c2[None, :]
    d = jnp.sqrt(jnp.maximum(d2, 0.0))
    out_ref[...] = jnp.argmin(d, axis=1).astype(jnp.int32)


@jax.jit
def kernel(x, cluster_centers):
    return pl.pallas_call(
        _assign_kernel,
        grid=(N // BN,),
        in_specs=[
            pl.BlockSpec((BN, DIM), lambda i: (i, 0)),
            pl.BlockSpec((K, DIM), lambda i: (0, 0)),
        ],
        out_specs=pl.BlockSpec((BN,), lambda i: (i,)),
        out_shape=jax.ShapeDtypeStruct((N,), jnp.int32),
    )(x, cluster_centers)


# speedup vs baseline: 1.3183x; 1.3183x over previous
import jax
import jax.numpy as jnp
from jax.experimental import pallas as pl

N = 32768
DIM = 64
K = 1024
BN = 2048


def _assign_kernel(x_ref, c_ref, out_ref):
    x = x_ref[...]
    c = c_ref[...]
    c2 = jnp.sum(c * c, axis=1)
    cneg = c * (-2.0)
    dot = jax.lax.dot_general(
        x, cneg, (((1,), (1,)), ((), ())),
        preferred_element_type=jnp.float32)
    s = dot + c2[None, :]
    out_ref[...] = jnp.argmin(s, axis=1).astype(jnp.int32)


@jax.jit
def kernel(x, cluster_centers):
    return pl.pallas_call(
        _assign_kernel,
        grid=(N // BN,),
        in_specs=[
            pl.BlockSpec((BN, DIM), lambda i: (i, 0)),
            pl.BlockSpec((K, DIM), lambda i: (0, 0)),
        ],
        out_specs=pl.BlockSpec((BN,), lambda i: (i,)),
        out_shape=jax.ShapeDtypeStruct((N,), jnp.int32),
    )(x, cluster_centers)


# speedup vs baseline: 2.7843x; 2.1120x over previous
import jax
import jax.numpy as jnp
from jax.experimental import pallas as pl

N = 32768
DIM = 64
K = 1024
BN = 2048


def _assign_kernel(x_ref, c_ref, out_ref):
    x = x_ref[...]
    c = c_ref[...]
    c2 = jnp.sum(c * c, axis=1)
    cneg = c * (-2.0)
    dot = jax.lax.dot_general(
        cneg, x, (((1,), (1,)), ((), ())),
        preferred_element_type=jnp.float32)
    s = dot + c2[:, None]
    out_ref[...] = jnp.argmin(s, axis=0).astype(jnp.int32)


@jax.jit
def kernel(x, cluster_centers):
    return pl.pallas_call(
        _assign_kernel,
        grid=(N // BN,),
        in_specs=[
            pl.BlockSpec((BN, DIM), lambda i: (i, 0)),
            pl.BlockSpec((K, DIM), lambda i: (0, 0)),
        ],
        out_specs=pl.BlockSpec((BN,), lambda i: (i,)),
        out_shape=jax.ShapeDtypeStruct((N,), jnp.int32),
    )(x, cluster_centers)


# speedup vs baseline: 2.9007x; 1.0418x over previous
import jax
import jax.numpy as jnp
from jax.experimental import pallas as pl

N = 32768
DIM = 64
K = 1024
BN = 4096


def _assign_kernel(x_ref, c_ref, out_ref):
    x = x_ref[...]
    c = c_ref[...]
    c2 = jnp.sum(c * c, axis=1)
    cneg = c * (-2.0)
    dot = jax.lax.dot_general(
        cneg, x, (((1,), (1,)), ((), ())),
        preferred_element_type=jnp.float32)
    s = dot + c2[:, None]
    out_ref[...] = jnp.argmin(s, axis=0).astype(jnp.int32)


@jax.jit
def kernel(x, cluster_centers):
    return pl.pallas_call(
        _assign_kernel,
        grid=(N // BN,),
        in_specs=[
            pl.BlockSpec((BN, DIM), lambda i: (i, 0)),
            pl.BlockSpec((K, DIM), lambda i: (0, 0)),
        ],
        out_specs=pl.BlockSpec((BN,), lambda i: (i,)),
        out_shape=jax.ShapeDtypeStruct((N,), jnp.int32),
    )(x, cluster_centers)


# speedup vs baseline: 2.9199x; 1.0066x over previous
import jax
import jax.numpy as jnp
from jax.experimental import pallas as pl

N = 32768
DIM = 64
K = 1024
BN = 8192


def _assign_kernel(x_ref, c_ref, out_ref):
    x = x_ref[...]
    c = c_ref[...]
    c2 = jnp.sum(c * c, axis=1)
    cneg = c * (-2.0)
    dot = jax.lax.dot_general(
        cneg, x, (((1,), (1,)), ((), ())),
        preferred_element_type=jnp.float32)
    s = dot + c2[:, None]
    out_ref[...] = jnp.argmin(s, axis=0).astype(jnp.int32)


@jax.jit
def kernel(x, cluster_centers):
    return pl.pallas_call(
        _assign_kernel,
        grid=(N // BN,),
        in_specs=[
            pl.BlockSpec((BN, DIM), lambda i: (i, 0)),
            pl.BlockSpec((K, DIM), lambda i: (0, 0)),
        ],
        out_specs=pl.BlockSpec((BN,), lambda i: (i,)),
        out_shape=jax.ShapeDtypeStruct((N,), jnp.int32),
    )(x, cluster_centers)


# speedup vs baseline: 3.0086x; 1.0304x over previous
import jax
import jax.numpy as jnp
from jax.experimental import pallas as pl

N = 32768
DIM = 64
K = 1024
BN = 8192


def _assign_kernel(x_ref, c_ref, out_ref):
    x = x_ref[...]
    c = c_ref[...]
    c2 = jnp.sum(c * c, axis=1)
    ca = jnp.concatenate([c * (-2.0), c2[:, None]], axis=1)
    xa = jnp.concatenate([x, jnp.ones((BN, 1), jnp.float32)], axis=1)
    s = jax.lax.dot_general(
        ca, xa, (((1,), (1,)), ((), ())),
        preferred_element_type=jnp.float32)
    out_ref[...] = jnp.argmin(s, axis=0).astype(jnp.int32)


@jax.jit
def kernel(x, cluster_centers):
    return pl.pallas_call(
        _assign_kernel,
        grid=(N // BN,),
        in_specs=[
            pl.BlockSpec((BN, DIM), lambda i: (i, 0)),
            pl.BlockSpec((K, DIM), lambda i: (0, 0)),
        ],
        out_specs=pl.BlockSpec((BN,), lambda i: (i,)),
        out_shape=jax.ShapeDtypeStruct((N,), jnp.int32),
    )(x, cluster_centers)
